# baseline (speedup 1.0000x reference)
import functools

import jax
import jax.numpy as jnp
import numpy as np
from jax import lax
from jax.experimental import pallas as pl
from jax.experimental.pallas import tpu as pltpu
from jax.experimental.pallas import tpu_sc as plsc

DIM = 4096
ROWS = 2 * 4096
NC = 2
NS = 16
L = 16
NW = NC * NS
ROWS_PER_W = ROWS // NW
R = 8
SLABS = ROWS_PER_W // R
HD = DIM // 2
JV = DIM // L
JH = JV // 2

_SIGN = np.int32(-(2 ** 31))
_IDX = np.int32(DIM - 1)

_mesh = plsc.VectorSubcoreMesh(core_axis_name="c", subcore_axis_name="s")


@functools.partial(
    pl.kernel,
    mesh=_mesh,
    compiler_params=pltpu.CompilerParams(needs_layout_passes=False),
    out_type=jax.ShapeDtypeStruct((ROWS, DIM), jnp.float32),
    scratch_types=[
        pltpu.VMEM((DIM,), jnp.int32),
        pltpu.VMEM((DIM,), jnp.float32),
        pltpu.VMEM((DIM,), jnp.float32),
        pltpu.VMEM((R, DIM), jnp.float32),
        pltpu.VMEM((R, DIM), jnp.float32),
        pltpu.VMEM((R, HD), jnp.float32),
        pltpu.VMEM((R, HD), jnp.float32),
        pltpu.SemaphoreType.DMA,
        pltpu.SemaphoreType.DMA,
        pltpu.SemaphoreType.DMA,
        pltpu.SemaphoreType.DMA,
    ],
)
def _dpd_sc(x_hbm, s1_hbm, s2_hbm, perm_hbm, out_hbm,
            perm_v, s1_v, s2_v, in0, in1, outh0, outh1,
            sem_i0, sem_i1, sem_o0, sem_o1):
    wid = lax.axis_index("s") * NC + lax.axis_index("c")
    row0 = wid * ROWS_PER_W

    pltpu.sync_copy(perm_hbm, perm_v)
    pltpu.sync_copy(s1_hbm, s1_v)
    pltpu.sync_copy(s2_hbm, s2_v)

    @plsc.parallel_loop(0, JV, unroll=4)
    def _sign_loop(j):
        sl = pl.ds(j * L, L)
        pv = perm_v[sl]
        b1 = plsc.bitcast(plsc.load_gather(s1_v, [pv]), jnp.int32)
        b2 = plsc.bitcast(s2_v[sl], jnp.int32)
        perm_v[sl] = pv | ((b1 ^ b2) & _SIGN)

    def start_in(slot, sem, s):
        pltpu.async_copy(x_hbm.at[pl.ds(row0 + s * R, R)], slot, sem)

    def wait_in(slot, sem):
        pltpu.make_async_copy(x_hbm.at[pl.ds(row0, R)], slot, sem).wait()

    def start_out(slot, sem, s, h):
        pltpu.async_copy(
            slot, out_hbm.at[pl.ds(row0 + s * R, R), pl.ds(h * HD, HD)], sem)

    def wait_out(slot, sem):
        pltpu.make_async_copy(
            slot, out_hbm.at[pl.ds(row0, R), pl.ds(0, HD)], sem).wait()

    def compute_half(in_ref, out_ref, h):
        @plsc.parallel_loop(h * JH, (h + 1) * JH, unroll=16)
        def _jloop(j):
            sl = pl.ds(j * L, L)
            pk = perm_v[sl]
            m = pk & _SIGN
            b = pk & _IDX
            co = j * L - h * HD
            for r in range(R):
                ri = jnp.full((L,), r, dtype=jnp.int32)
                g = plsc.load_gather(in_ref, [ri, b])
                gi = plsc.bitcast(g, jnp.int32) ^ m
                out_ref[r, pl.ds(co, L)] = plsc.bitcast(gi, jnp.float32)

    start_in(in0, sem_i0, 0)
    start_in(in1, sem_i1, 1)

    T = SLABS // 2

    def process_slab(in_ref, sem_i, s, first):
        wait_in(in_ref, sem_i)

        @pl.when(jnp.logical_not(first))
        def _():
            wait_out(outh0, sem_o0)

        compute_half(in_ref, outh0, 0)
        start_out(outh0, sem_o0, s, 0)

        @pl.when(jnp.logical_not(first))
        def _():
            wait_out(outh1, sem_o1)

        compute_half(in_ref, outh1, 1)
        start_out(outh1, sem_o1, s, 1)

    def cbody(t, carry):
        process_slab(in0, sem_i0, 2 * t, t == 0)

        @pl.when(t < T - 1)
        def _():
            start_in(in0, sem_i0, 2 * t + 2)

        process_slab(in1, sem_i1, 2 * t + 1, jnp.bool_(False))

        @pl.when(t < T - 1)
        def _():
            start_in(in1, sem_i1, 2 * t + 3)

        return carry

    lax.fori_loop(0, T, cbody, 0)

    wait_out(outh0, sem_o0)
    wait_out(outh1, sem_o1)


def kernel(x, sign1, sign2, perm):
    out = _dpd_sc(x.reshape(ROWS, DIM), sign1, sign2, perm.astype(jnp.int32))
    return out.reshape(x.shape)


# speedup vs baseline: 1.1796x; 1.1796x over previous
import functools

import jax
import jax.numpy as jnp
import numpy as np
from jax import lax
from jax.experimental import pallas as pl
from jax.experimental.pallas import tpu as pltpu
from jax.experimental.pallas import tpu_sc as plsc

DIM = 4096
ROWS = 2 * 4096
NC = 2
NS = 16
L = 16
NW = NC * NS
ROWS_PER_W = ROWS // NW
R = 8
SLABS = ROWS_PER_W // R
HD = DIM // 2
JV = DIM // L
JH = JV // 2

_SIGN = np.int32(-(2 ** 31))
_IDX = np.int32(DIM - 1)

_mesh = plsc.VectorSubcoreMesh(core_axis_name="c", subcore_axis_name="s")


@functools.partial(
    pl.kernel,
    mesh=_mesh,
    compiler_params=pltpu.CompilerParams(needs_layout_passes=False),
    out_type=jax.ShapeDtypeStruct((ROWS, DIM), jnp.float32),
    scratch_types=[
        pltpu.VMEM((DIM,), jnp.int32),
        pltpu.VMEM((DIM,), jnp.float32),
        pltpu.VMEM((DIM,), jnp.float32),
        pltpu.VMEM((R, DIM), jnp.float32),
        pltpu.VMEM((R, DIM), jnp.float32),
        pltpu.VMEM((R, HD), jnp.float32),
        pltpu.VMEM((R, HD), jnp.float32),
        pltpu.VMEM((R, HD), jnp.float32),
        pltpu.SemaphoreType.DMA,
        pltpu.SemaphoreType.DMA,
        pltpu.SemaphoreType.DMA,
        pltpu.SemaphoreType.DMA,
        pltpu.SemaphoreType.DMA,
    ],
)
def _dpd_sc(x_hbm, s1_hbm, s2_hbm, perm_hbm, out_hbm,
            perm_v, s1_v, s2_v, in0, in1, outh0, outh1, outh2,
            sem_i0, sem_i1, sem_o0, sem_o1, sem_o2):
    wid = lax.axis_index("s") * NC + lax.axis_index("c")
    row0 = wid * ROWS_PER_W

    pltpu.async_copy(x_hbm.at[pl.ds(row0, R)], in0, sem_i0)
    pltpu.async_copy(x_hbm.at[pl.ds(row0 + R, R)], in1, sem_i1)

    pltpu.sync_copy(perm_hbm, perm_v)
    pltpu.sync_copy(s1_hbm, s1_v)
    pltpu.sync_copy(s2_hbm, s2_v)

    @plsc.parallel_loop(0, JV, unroll=4)
    def _sign_loop(j):
        sl = pl.ds(j * L, L)
        pv = perm_v[sl]
        b1 = plsc.bitcast(plsc.load_gather(s1_v, [pv]), jnp.int32)
        b2 = plsc.bitcast(s2_v[sl], jnp.int32)
        perm_v[sl] = pv | ((b1 ^ b2) & _SIGN)

    def start_in(slot, sem, s):
        pltpu.async_copy(x_hbm.at[pl.ds(row0 + s * R, R)], slot, sem)

    def wait_in(slot, sem):
        pltpu.make_async_copy(x_hbm.at[pl.ds(row0, R)], slot, sem).wait()

    def start_out(slot, sem, s, h):
        pltpu.async_copy(
            slot, out_hbm.at[pl.ds(row0 + s * R, R), pl.ds(h * HD, HD)], sem)

    def wait_out(slot, sem):
        pltpu.make_async_copy(
            slot, out_hbm.at[pl.ds(row0, R), pl.ds(0, HD)], sem).wait()

    ris = [jnp.full((L,), r, dtype=jnp.int32) for r in range(R)]

    def compute_half(in_ref, out_ref, h):
        @plsc.parallel_loop(h * JH, (h + 1) * JH, unroll=8)
        def _jloop(j):
            sl = pl.ds(j * L, L)
            pk = perm_v[sl]
            m = pk & _SIGN
            b = pk & _IDX
            co = j * L - h * HD
            for r in range(R):
                g = plsc.load_gather(in_ref, [ris[r], b])
                gi = plsc.bitcast(g, jnp.int32) ^ m
                out_ref[r, pl.ds(co, L)] = plsc.bitcast(gi, jnp.float32)

    T = SLABS // 2

    def process_slab(in_ref, sem_i, s, h0_slot, h0_sem, first_h0, first_h1):
        wait_in(in_ref, sem_i)

        @pl.when(jnp.logical_not(first_h0))
        def _():
            wait_out(h0_slot, h0_sem)

        compute_half(in_ref, h0_slot, 0)
        start_out(h0_slot, h0_sem, s, 0)

        @pl.when(jnp.logical_not(first_h1))
        def _():
            wait_out(outh2, sem_o2)

        compute_half(in_ref, outh2, 1)
        start_out(outh2, sem_o2, s, 1)

    def cbody(t, carry):
        process_slab(in0, sem_i0, 2 * t, outh0, sem_o0, t == 0, t == 0)

        @pl.when(t < T - 1)
        def _():
            start_in(in0, sem_i0, 2 * t + 2)

        process_slab(in1, sem_i1, 2 * t + 1, outh1, sem_o1, t == 0,
                     jnp.bool_(False))

        @pl.when(t < T - 1)
        def _():
            start_in(in1, sem_i1, 2 * t + 3)

        return carry

    lax.fori_loop(0, T, cbody, 0)

    wait_out(outh0, sem_o0)
    wait_out(outh1, sem_o1)
    wait_out(outh2, sem_o2)


def kernel(x, sign1, sign2, perm):
    out = _dpd_sc(x.reshape(ROWS, DIM), sign1, sign2, perm.astype(jnp.int32))
    return out.reshape(x.shape)


# speedup vs baseline: 1.1823x; 1.0023x over previous
import functools

import jax
import jax.numpy as jnp
import numpy as np
from jax import lax
from jax.experimental import pallas as pl
from jax.experimental.pallas import tpu as pltpu
from jax.experimental.pallas import tpu_sc as plsc

DIM = 4096
ROWS = 2 * 4096
NC = 2
NS = 16
L = 16
NW = NC * NS
ROWS_PER_W = ROWS // NW
R = 8
SLABS = ROWS_PER_W // R
HD = DIM // 2
JV = DIM // L
JH = JV // 2

_SIGN = np.int32(-(2 ** 31))
_IDX = np.int32(DIM - 1)

_mesh = plsc.VectorSubcoreMesh(core_axis_name="c", subcore_axis_name="s")


@functools.partial(
    pl.kernel,
    mesh=_mesh,
    compiler_params=pltpu.CompilerParams(
        needs_layout_passes=False, skip_device_barrier=True),
    out_type=jax.ShapeDtypeStruct((ROWS, DIM), jnp.float32),
    scratch_types=[
        pltpu.VMEM((DIM,), jnp.int32),
        pltpu.VMEM((DIM,), jnp.float32),
        pltpu.VMEM((DIM,), jnp.float32),
        pltpu.VMEM((R, DIM), jnp.float32),
        pltpu.VMEM((R, DIM), jnp.float32),
        pltpu.VMEM((R, HD), jnp.float32),
        pltpu.VMEM((R, HD), jnp.float32),
        pltpu.VMEM((R, HD), jnp.float32),
        pltpu.SemaphoreType.DMA,
        pltpu.SemaphoreType.DMA,
        pltpu.SemaphoreType.DMA,
        pltpu.SemaphoreType.DMA,
        pltpu.SemaphoreType.DMA,
    ],
)
def _dpd_sc(x_hbm, s1_hbm, s2_hbm, perm_hbm, out_hbm,
            perm_v, s1_v, s2_v, in0, in1, outh0, outh1, outh2,
            sem_i0, sem_i1, sem_o0, sem_o1, sem_o2):
    wid = lax.axis_index("s") * NC + lax.axis_index("c")
    row0 = wid * ROWS_PER_W

    pltpu.async_copy(x_hbm.at[pl.ds(row0, R)], in0, sem_i0)
    pltpu.async_copy(x_hbm.at[pl.ds(row0 + R, R)], in1, sem_i1)

    pltpu.sync_copy(perm_hbm, perm_v)
    pltpu.sync_copy(s1_hbm, s1_v)
    pltpu.sync_copy(s2_hbm, s2_v)

    @plsc.parallel_loop(0, JV, unroll=4)
    def _sign_loop(j):
        sl = pl.ds(j * L, L)
        pv = perm_v[sl]
        b1 = plsc.bitcast(plsc.load_gather(s1_v, [pv]), jnp.int32)
        b2 = plsc.bitcast(s2_v[sl], jnp.int32)
        perm_v[sl] = pv | ((b1 ^ b2) & _SIGN)

    def start_in(slot, sem, s):
        pltpu.async_copy(x_hbm.at[pl.ds(row0 + s * R, R)], slot, sem)

    def wait_in(slot, sem):
        pltpu.make_async_copy(x_hbm.at[pl.ds(row0, R)], slot, sem).wait()

    def start_out(slot, sem, s, h):
        pltpu.async_copy(
            slot, out_hbm.at[pl.ds(row0 + s * R, R), pl.ds(h * HD, HD)], sem)

    def wait_out(slot, sem):
        pltpu.make_async_copy(
            slot, out_hbm.at[pl.ds(row0, R), pl.ds(0, HD)], sem).wait()

    ris = [jnp.full((L,), r, dtype=jnp.int32) for r in range(R)]

    def compute_half(in_ref, out_ref, h):
        @plsc.parallel_loop(h * JH, (h + 1) * JH, unroll=8)
        def _jloop(j):
            sl = pl.ds(j * L, L)
            pk = perm_v[sl]
            m = pk & _SIGN
            b = pk & _IDX
            co = j * L - h * HD
            for r in range(R):
                g = plsc.load_gather(in_ref, [ris[r], b])
                gi = plsc.bitcast(g, jnp.int32) ^ m
                out_ref[r, pl.ds(co, L)] = plsc.bitcast(gi, jnp.float32)

    T = SLABS // 2

    def process_slab(in_ref, sem_i, s, h0_slot, h0_sem, first_h0, first_h1):
        wait_in(in_ref, sem_i)

        @pl.when(jnp.logical_not(first_h0))
        def _():
            wait_out(h0_slot, h0_sem)

        compute_half(in_ref, h0_slot, 0)
        start_out(h0_slot, h0_sem, s, 0)

        @pl.when(jnp.logical_not(first_h1))
        def _():
            wait_out(outh2, sem_o2)

        compute_half(in_ref, outh2, 1)
        start_out(outh2, sem_o2, s, 1)

    def cbody(t, carry):
        process_slab(in0, sem_i0, 2 * t, outh0, sem_o0, t == 0, t == 0)

        @pl.when(t < T - 1)
        def _():
            start_in(in0, sem_i0, 2 * t + 2)

        process_slab(in1, sem_i1, 2 * t + 1, outh1, sem_o1, t == 0,
                     jnp.bool_(False))

        @pl.when(t < T - 1)
        def _():
            start_in(in1, sem_i1, 2 * t + 3)

        return carry

    lax.fori_loop(0, T, cbody, 0)

    wait_out(outh0, sem_o0)
    wait_out(outh1, sem_o1)
    wait_out(outh2, sem_o2)


def kernel(x, sign1, sign2, perm):
    out = _dpd_sc(x.reshape(ROWS, DIM), sign1, sign2, perm.astype(jnp.int32))
    return out.reshape(x.shape)
